# baseline (speedup 1.0000x reference)
import functools

import jax
import jax.numpy as jnp
from jax import lax
from jax.experimental import pallas as pl
from jax.experimental.pallas import tpu as pltpu
from jax.experimental.pallas import tpu_sc as plsc

B, L, V, E, H = 1024, 50, 1000000, 64, 64
G4 = 4 * H


def _lstm_body(e_ref, wih_ref, whh_ref, b_ref, out_hbm,
               h_ref, c_ref, h_buf, out_sem):
    t = pl.program_id(0)

    @pl.when(t == 0)
    def _():
        h_ref[...] = jnp.zeros((B, H), jnp.float32)
        c_ref[...] = jnp.zeros((B, H), jnp.float32)

    h = h_ref[...]
    c = c_ref[...]
    et = e_ref[...]

    def gate(g):
        w_i = wih_ref[pl.ds(g * E, E), :]
        w_h = whh_ref[pl.ds(g * H, H), :]
        acc = jnp.dot(et, w_i, preferred_element_type=jnp.float32)
        acc += jnp.dot(h, w_h, preferred_element_type=jnp.float32)
        return acc + b_ref[g, :]

    i = jax.nn.sigmoid(gate(0))
    f = jax.nn.sigmoid(gate(1))
    g = jnp.tanh(gate(2))
    o = jax.nn.sigmoid(gate(3))
    c = f * c + i * g
    h = o * jnp.tanh(c)
    h_ref[...] = h
    c_ref[...] = c

    def out_copy(tt, slot):
        return pltpu.make_async_copy(
            h_buf.at[slot], out_hbm.at[:, tt], out_sem.at[slot]
        )

    slot = lax.rem(t, 2)

    @pl.when(t >= 2)
    def _():
        out_copy(t - 2, slot).wait()

    h_buf[slot] = h
    out_copy(t, slot).start()

    @pl.when(t == L - 1)
    def _():
        out_copy(t - 1, lax.rem(t - 1, 2)).wait()
        out_copy(t, slot).wait()


def _lstm_tc(e_flat, wih_s, whh_s, bias4):
    return pl.pallas_call(
        _lstm_body,
        grid=(L,),
        in_specs=[
            pl.BlockSpec((B, E), lambda t: (t, 0)),
            pl.BlockSpec((G4, H), lambda t: (0, 0)),
            pl.BlockSpec((G4, H), lambda t: (0, 0)),
            pl.BlockSpec((4, H), lambda t: (0, 0)),
        ],
        out_specs=pl.BlockSpec(memory_space=pl.ANY),
        out_shape=jax.ShapeDtypeStruct((B, L, H), jnp.float32),
        scratch_shapes=[
            pltpu.VMEM((B, H), jnp.float32),
            pltpu.VMEM((B, H), jnp.float32),
            pltpu.VMEM((2, B, H), jnp.float32),
            pltpu.SemaphoreType.DMA((2,)),
        ],
    )(e_flat, wih_s, whh_s, bias4)


def kernel(x, emb, W_ih, W_hh, b_ih, b_hh):
    e = jnp.take(emb, x.T.reshape(-1), axis=0)
    wih_s = W_ih.reshape(4, H, E).transpose(0, 2, 1).reshape(4 * E, H)
    whh_s = W_hh.reshape(4, H, H).transpose(0, 2, 1).reshape(4 * H, H)
    bias4 = (b_ih + b_hh).reshape(4, H)
    return _lstm_tc(e, wih_s, whh_s, bias4)
